# baseline (speedup 1.0000x reference)
import functools

import jax
import jax.numpy as jnp
from jax.experimental import pallas as pl

BM = 512


def _matmul_kernel(x_ref, w0_ref, w1_ref, o_ref):
    x = x_ref[...]
    o_ref[0] = jax.lax.dot_general(
        x, w0_ref[...], (((0,), (1,)), ((), ())),
        preferred_element_type=jnp.float32,
    )
    o_ref[1] = jax.lax.dot_general(
        x, w1_ref[...], (((0,), (1,)), ((), ())),
        preferred_element_type=jnp.float32,
    )


@functools.partial(jax.jit, static_argnames=())
def kernel(input, weight):
    m, k = weight.shape
    _, n = input.shape
    half = m // 2 // BM
    out_t = pl.pallas_call(
        _matmul_kernel,
        grid=(half,),
        in_specs=[
            pl.BlockSpec((k, n), lambda i: (0, 0)),
            pl.BlockSpec((BM, k), lambda i: (i, 0)),
            pl.BlockSpec((BM, k), lambda i: (half + i, 0)),
        ],
        out_specs=pl.BlockSpec((2, n, BM), lambda i: (0, 0, i)),
        out_shape=jax.ShapeDtypeStruct((2, n, m // 2), jnp.float32),
    )(input, weight, weight)
    return jnp.concatenate([out_t[0].T, out_t[1].T], axis=0)


# speedup vs baseline: 1.0263x; 1.0263x over previous
import functools

import jax
import jax.numpy as jnp
from jax.experimental import pallas as pl

BM = 512


def _matmul_kernel(x_ref, w_ref, o_ref):
    t = jax.lax.dot_general(
        x_ref[...],
        w_ref[...],
        (((0,), (1,)), ((), ())),
        preferred_element_type=jnp.float32,
    )
    n = t.shape[0]
    eye = jnp.eye(n, dtype=jnp.float32)
    o_ref[...] = jax.lax.dot_general(
        t, eye, (((0,), (0,)), ((), ())),
        preferred_element_type=jnp.float32,
    )


@functools.partial(jax.jit, static_argnames=())
def kernel(input, weight):
    m, k = weight.shape
    _, n = input.shape
    return pl.pallas_call(
        _matmul_kernel,
        grid=(m // BM,),
        in_specs=[
            pl.BlockSpec((k, n), lambda i: (0, 0)),
            pl.BlockSpec((BM, k), lambda i: (i, 0)),
        ],
        out_specs=pl.BlockSpec((BM, n), lambda i: (i, 0)),
        out_shape=jax.ShapeDtypeStruct((m, n), jnp.float32),
    )(input, weight)


# speedup vs baseline: 1.1151x; 1.0865x over previous
import functools

import jax
import jax.numpy as jnp
from jax.experimental import pallas as pl

BM = 1024


def _matmul_kernel(x_ref, w_ref, o_ref):
    o_ref[...] = jax.lax.dot_general(
        x_ref[...],
        w_ref[...],
        (((0,), (1,)), ((), ())),
        preferred_element_type=jnp.float32,
    )


@functools.partial(jax.jit, static_argnames=())
def kernel(input, weight):
    m, k = weight.shape
    _, n = input.shape
    out_t = pl.pallas_call(
        _matmul_kernel,
        grid=(m // BM,),
        in_specs=[
            pl.BlockSpec((k, n), lambda i: (0, 0)),
            pl.BlockSpec((BM, k), lambda i: (i, 0)),
        ],
        out_specs=pl.BlockSpec((n, BM), lambda i: (0, i)),
        out_shape=jax.ShapeDtypeStruct((n, m), jnp.float32),
    )(input, weight)
    return out_t.T
